# baseline (speedup 1.0000x reference)
import functools

import jax
import jax.numpy as jnp
from jax import lax
from jax.experimental import pallas as pl
from jax.experimental.pallas import tpu as pltpu
from jax.experimental.pallas import tpu_sc as plsc

_NC = 2
_NS = 16
_NW = _NC * _NS

_D = 64
_NR = 32
_PANELS = 245
_RNG = _PANELS * 128
_ICH = 4096
_DUMP = None


def _iota16():
    return lax.broadcasted_iota(jnp.int32, (16,), 0)


def _f16(v):
    return jnp.full((16,), v, jnp.int32)


def _sc_sweep_gather(table_t, tail_t, idx):
    n = table_t.shape[1]
    b3 = idx.shape[0]
    n_ich = b3 // _ICH
    kcap = b3 + 32

    mesh = plsc.VectorSubcoreMesh(core_axis_name="c", subcore_axis_name="s")

    @functools.partial(
        pl.kernel,
        out_type=jax.ShapeDtypeStruct((b3 + 1, 128), jnp.float32),
        mesh=mesh,
        compiler_params=pltpu.CompilerParams(use_tc_tiling_on_sc=True,
                                             needs_layout_passes=False),
        scratch_types=[
            pltpu.VMEM((_ICH,), jnp.int32),
            pltpu.VMEM((kcap,), jnp.int32),
            pltpu.VMEM((64, 128), jnp.float32),
            pltpu.VMEM((64, 128), jnp.float32),
            pltpu.VMEM((256, 128), jnp.float32),
            pltpu.VMEM((2, 128), jnp.int32),
            pltpu.VMEM((32,), jnp.int32),
            pltpu.SemaphoreType.DMA,
            pltpu.SemaphoreType.DMA,
            pltpu.SemaphoreType.DMA,
        ],
    )
    def sweep_kernel(table_hbm, tail_hbm, idx_hbm, out_hbm, idxbuf, keys,
                     panel_a, panel_b, rows, pos2d, tmpv,
                     sem_a, sem_b, sem_f):
        wid = lax.axis_index("s") * _NC + lax.axis_index("c")
        lo = wid * _RNG
        it16 = _iota16()
        keys_r = keys.at[pl.ds(0, kcap)]
        junk = b3 + 16

        def fchunk(cnt, c):
            pltpu.sync_copy(idx_hbm.at[pl.ds(c * _ICH, _ICH)], idxbuf)

            def fbody(k, cnt):
                v = idxbuf[pl.ds(k * 16, 16)]
                vl = v - lo
                m = (vl >= 0) & (vl < _RNG)
                pos = c * _ICH + k * 16 + it16
                key = ((vl >> 7) << 23) | ((vl & 127) << 16) | pos
                csum = plsc.cumsum(m.astype(jnp.int32))
                dst = jnp.where(m, cnt + csum - 1, junk)
                plsc.store_scatter(keys_r, [dst], key)
                return cnt + csum[15]

            return lax.fori_loop(0, _ICH // 16, fbody, cnt)

        cnt = jnp.int32(0)
        for c in range(n_ich):
            cnt = fchunk(cnt, c)
        plsc.store_scatter(keys_r, [cnt + it16], _f16(-1))

        for hh in range(2):
            for i in range(8):
                pos2d[hh, pl.ds(i * 16, 16)] = _f16(b3)

        def fire(t, panel_v, sem):
            g0 = lo + t * 128
            tlim = jnp.int32(t) < _PANELS
            full = tlim & (g0 + 128 <= n)
            tail = tlim & (g0 < n) & (g0 + 128 > n)

            @pl.when(full)
            def _():
                pltpu.make_async_copy(
                    table_hbm.at[:, pl.ds(g0, 128)], panel_v, sem).start()

            @pl.when(tail)
            def _():
                pltpu.make_async_copy(tail_hbm, panel_v, sem).start()

        def wait(t, panel_v, sem):
            g0 = lo + t * 128
            tlim = jnp.int32(t) < _PANELS
            full = tlim & (g0 + 128 <= n)
            tail = tlim & (g0 < n) & (g0 + 128 > n)

            @pl.when(full)
            def _():
                pltpu.make_async_copy(
                    table_hbm.at[:, pl.ds(g0, 128)], panel_v, sem).wait()

            @pl.when(tail)
            def _():
                pltpu.make_async_copy(tail_hbm, panel_v, sem).wait()

        def flush(flushed):
            hbit = (flushed >> 7) & 1
            for hh in range(2):
                @pl.when(hbit == hh)
                def _(hh=hh):
                    desc = pltpu.make_async_copy(
                        rows.at[pl.ds(hh * 128, 128)],
                        out_hbm.at[pos2d.at[hh]],
                        sem_f,
                    )
                    desc.start()
                    desc.wait()
                    for i in range(8):
                        pos2d[hh, pl.ds(i * 16, 16)] = _f16(b3)

        def process(t, panel_v, carry):
            def sbody(k, carry):
                scnt, flushed = carry
                v = keys[pl.ds(k * 16, 16)]
                m = (v >> 23) == t
                csum = plsc.cumsum(m.astype(jnp.int32))
                c = csum[15]

                need = (scnt + 16 - flushed) > 256

                @pl.when(need)
                def _():
                    flush(flushed)

                flushed = jnp.where(need, flushed + 128, flushed)

                @pl.when(c > 0)
                def _():
                    dst = jnp.where(m, csum - 1, 16)
                    plsc.store_scatter(tmpv.at[pl.ds(0, 32)], [dst],
                                       v & 0x7FFFFF)
                    cp = tmpv[pl.ds(0, 16)]
                    lm = it16 < c
                    colv = jnp.where(lm, (cp >> 16) & 127, 0)
                    posv = jnp.where(lm, cp & 0xFFFF, b3)
                    slotv = (scnt + it16) & 255
                    for f in range(64):
                        vals = plsc.load_gather(panel_v, [_f16(f), colv])
                        plsc.store_scatter(rows, [slotv, _f16(f)], vals)
                    plsc.store_scatter(pos2d, [slotv >> 7, slotv & 127],
                                       posv)

                return (scnt + c, flushed)

            return lax.fori_loop(0, (cnt + 15) >> 4, sbody, carry)

        fire(0, panel_a, sem_a)
        fire(1, panel_b, sem_b)

        def outer(t2, carry):
            t = t2 * 2
            wait(t, panel_a, sem_a)
            carry = process(t, panel_a, carry)
            fire(t + 2, panel_a, sem_a)
            wait(t + 1, panel_b, sem_b)
            carry = process(t + 1, panel_b, carry)
            fire(t + 3, panel_b, sem_b)
            return carry

        scnt, flushed = lax.fori_loop(0, _PANELS // 2, outer,
                                      (jnp.int32(0), jnp.int32(0)))
        t_last = _PANELS - 1
        wait(t_last, panel_a, sem_a)
        scnt, flushed = process(t_last, panel_a, (scnt, flushed))

        for _ in range(2):
            need = flushed < scnt

            @pl.when(need)
            def _():
                flush(flushed)

            flushed = jnp.where(need, flushed + 128, flushed)

    return sweep_kernel(table_t, tail_t, idx)


def _tc_transform(r2d, rows3, w_flat, rel_embed, batch, blk):
    n_blocks = batch // blk
    kdim = _NR * _D

    def body(r_ref, xh_ref, xp_ref, xn_ref, wf_ref, rel_ref,
             oh_ref, op_ref, on_ref, or_ref):
        rcol = r_ref[...]
        lane_rel = lax.broadcasted_iota(jnp.int32, (blk, kdim), 1) >> 6
        mask = lane_rel == rcol
        wf = wf_ref[...]

        def trans(x_ref, o_ref):
            x = x_ref[...][:, :_D]
            xt = jnp.concatenate([x] * _NR, axis=1)
            z = jnp.where(mask, xt, 0.0)
            o_ref[...] = jnp.dot(z, wf, preferred_element_type=jnp.float32)

        trans(xh_ref, oh_ref)
        trans(xp_ref, op_ref)
        trans(xn_ref, on_ref)

        onehot = (lax.broadcasted_iota(jnp.int32, (blk, _NR), 1)
                  == rcol).astype(jnp.float32)
        or_ref[...] = jnp.dot(onehot, rel_ref[...],
                              preferred_element_type=jnp.float32)

    out_block = pl.BlockSpec((blk, _D), lambda i: (i, 0))
    return pl.pallas_call(
        body,
        grid=(n_blocks,),
        in_specs=[
            pl.BlockSpec((blk, 1), lambda i: (i, 0)),
            pl.BlockSpec((blk, 128), lambda i: (i, 0)),
            pl.BlockSpec((blk, 128), lambda i: (i + n_blocks, 0)),
            pl.BlockSpec((blk, 128), lambda i: (i + 2 * n_blocks, 0)),
            pl.BlockSpec((kdim, _D), lambda i: (0, 0)),
            pl.BlockSpec((_NR, _D), lambda i: (0, 0)),
        ],
        out_specs=[out_block, out_block, out_block, out_block],
        out_shape=[jax.ShapeDtypeStruct((batch, _D), jnp.float32)] * 4,
    )(r2d, rows3, rows3, rows3, w_flat, rel_embed)


def kernel(h, r, pos_t, neg_t, entity_embed, relation_embed, W_r):
    batch = h.shape[0]
    n_ent = entity_embed.shape[0]
    table_t = entity_embed.T
    tail = jnp.zeros((64, 128), jnp.float32)
    tail_n = n_ent - (n_ent // 128) * 128
    if tail_n:
        tail = tail.at[:, :tail_n].set(entity_embed[-tail_n:].T)
    idx_all = jnp.concatenate([h, pos_t, neg_t]).astype(jnp.int32)
    rows3 = _sc_sweep_gather(table_t, tail, idx_all)
    w_flat = W_r.reshape(_NR * _D, _D)
    r2d = r.astype(jnp.int32)[:, None]
    h_e, pos_t_e, neg_t_e, r_embed = _tc_transform(
        r2d, rows3, w_flat, relation_embed, batch, blk=512)
    return (h_e, pos_t_e, neg_t_e, r_embed)


# speedup vs baseline: 1.4358x; 1.4358x over previous
import functools

import jax
import jax.numpy as jnp
from jax import lax
from jax.experimental import pallas as pl
from jax.experimental.pallas import tpu as pltpu
from jax.experimental.pallas import tpu_sc as plsc

_NC = 2
_NS = 16
_NW = _NC * _NS

_D = 64
_NR = 32
_CHUNK = 128


def _sc_gather128(table128, idx):
    b3 = idx.shape[0]
    b_per_w = b3 // _NW
    n_q = 4
    q_rows = b_per_w // n_q
    n_chunks = q_rows // _CHUNK
    assert q_rows % _CHUNK == 0

    mesh = plsc.VectorSubcoreMesh(core_axis_name="c", subcore_axis_name="s")

    @functools.partial(
        pl.kernel,
        out_type=jax.ShapeDtypeStruct((b3, 128), jnp.float32),
        mesh=mesh,
        compiler_params=pltpu.CompilerParams(use_tc_tiling_on_sc=True),
        scratch_types=[
            pltpu.VMEM((b_per_w,), jnp.int32),
            pltpu.VMEM((q_rows, 128), jnp.float32),
            pltpu.VMEM((q_rows, 128), jnp.float32),
            pltpu.SemaphoreType.DMA,
            pltpu.SemaphoreType.DMA,
        ],
    )
    def gather_kernel(table_hbm, idx_hbm, out_hbm, idx_v, rows_a, rows_b,
                      sem_a, sem_b):
        wid = lax.axis_index("s") * _NC + lax.axis_index("c")
        base = wid * b_per_w
        pltpu.sync_copy(idx_hbm.at[pl.ds(base, b_per_w)], idx_v)
        bufs = ((rows_a, sem_a), (rows_b, sem_b))

        def chunk_copies(qq):
            rows_v, sem = bufs[qq % 2]
            return [
                pltpu.make_async_copy(
                    table_hbm.at[idx_v.at[pl.ds(qq * q_rows + j * _CHUNK,
                                                _CHUNK)]],
                    rows_v.at[pl.ds(j * _CHUNK, _CHUNK)],
                    sem,
                )
                for j in range(n_chunks)
            ]

        for c in chunk_copies(0):
            c.start()
        for qq in range(1, n_q + 1):
            if qq <= n_q - 1:
                for c in chunk_copies(qq):
                    c.start()
            prev = qq - 1
            rows_v, _ = bufs[prev % 2]
            for c in chunk_copies(prev):
                c.wait()
            pltpu.sync_copy(rows_v,
                            out_hbm.at[pl.ds(base + prev * q_rows, q_rows)])

    return gather_kernel(table128, idx)


def _tc_pad_transpose(table_t):
    d, n = table_t.shape
    blk = 512
    n_blocks = (n + blk - 1) // blk

    def body(x_ref, o_ref):
        x = x_ref[...]
        eye = (lax.broadcasted_iota(jnp.int32, (d, d), 0)
               == lax.broadcasted_iota(jnp.int32, (d, d), 1)
               ).astype(jnp.float32)
        xt = lax.dot_general(x, eye, (((0,), (0,)), ((), ())),
                             preferred_element_type=jnp.float32)
        o_ref[...] = jnp.concatenate(
            [xt, jnp.zeros((blk, 128 - d), jnp.float32)], axis=1)

    return pl.pallas_call(
        body,
        grid=(n_blocks,),
        in_specs=[pl.BlockSpec((d, blk), lambda i: (0, i))],
        out_specs=pl.BlockSpec((blk, 128), lambda i: (i, 0)),
        out_shape=jax.ShapeDtypeStruct((n, 128), jnp.float32),
    )(table_t)


def _tc_transform(r2d, rows3, w_flat, rel_embed, batch, blk):
    n_blocks = batch // blk
    kdim = _NR * _D

    def body(r_ref, xh_ref, xp_ref, xn_ref, wf_ref, rel_ref,
             oh_ref, op_ref, on_ref, or_ref):
        rcol = r_ref[...]
        lane_rel = lax.broadcasted_iota(jnp.int32, (blk, kdim), 1) >> 6
        mask = lane_rel == rcol
        wf = wf_ref[...]

        def trans(x_ref, o_ref):
            x = x_ref[...][:, :_D]
            xt = jnp.concatenate([x] * _NR, axis=1)
            z = jnp.where(mask, xt, 0.0)
            o_ref[...] = jnp.dot(z, wf, preferred_element_type=jnp.float32)

        trans(xh_ref, oh_ref)
        trans(xp_ref, op_ref)
        trans(xn_ref, on_ref)

        onehot = (lax.broadcasted_iota(jnp.int32, (blk, _NR), 1)
                  == rcol).astype(jnp.float32)
        or_ref[...] = jnp.dot(onehot, rel_ref[...],
                              preferred_element_type=jnp.float32)

    out_block = pl.BlockSpec((blk, _D), lambda i: (i, 0))
    return pl.pallas_call(
        body,
        grid=(n_blocks,),
        in_specs=[
            pl.BlockSpec((blk, 1), lambda i: (i, 0)),
            pl.BlockSpec((blk, 128), lambda i: (i, 0)),
            pl.BlockSpec((blk, 128), lambda i: (i + n_blocks, 0)),
            pl.BlockSpec((blk, 128), lambda i: (i + 2 * n_blocks, 0)),
            pl.BlockSpec((kdim, _D), lambda i: (0, 0)),
            pl.BlockSpec((_NR, _D), lambda i: (0, 0)),
        ],
        out_specs=[out_block, out_block, out_block, out_block],
        out_shape=[jax.ShapeDtypeStruct((batch, _D), jnp.float32)] * 4,
    )(r2d, rows3, rows3, rows3, w_flat, rel_embed)


def kernel(h, r, pos_t, neg_t, entity_embed, relation_embed, W_r):
    batch = h.shape[0]
    table128 = _tc_pad_transpose(entity_embed.T)
    idx_all = jnp.concatenate([h, pos_t, neg_t]).astype(jnp.int32)
    rows3 = _sc_gather128(table128, idx_all)
    w_flat = W_r.reshape(_NR * _D, _D)
    r2d = r.astype(jnp.int32)[:, None]
    h_e, pos_t_e, neg_t_e, r_embed = _tc_transform(
        r2d, rows3, w_flat, relation_embed, batch, blk=512)
    return (h_e, pos_t_e, neg_t_e, r_embed)


# speedup vs baseline: 3.0867x; 2.1497x over previous
import functools

import jax
import jax.numpy as jnp
from jax import lax
from jax.experimental import pallas as pl
from jax.experimental.pallas import tpu as pltpu
from jax.experimental.pallas import tpu_sc as plsc

_NC = 2
_NS = 16
_NW = _NC * _NS

_D = 64
_NR = 32
_CHUNK = 128


def _sc_gather128(table128, idx):
    b3 = idx.shape[0]
    b_per_w = b3 // _NW
    n_q = 4
    q_rows = b_per_w // n_q
    n_chunks = q_rows // _CHUNK
    assert q_rows % _CHUNK == 0

    mesh = plsc.VectorSubcoreMesh(core_axis_name="c", subcore_axis_name="s")

    @functools.partial(
        pl.kernel,
        out_type=jax.ShapeDtypeStruct((b3, 128), jnp.float32),
        mesh=mesh,
        compiler_params=pltpu.CompilerParams(use_tc_tiling_on_sc=True),
        scratch_types=[
            pltpu.VMEM((b_per_w,), jnp.int32),
            pltpu.VMEM((q_rows, 128), jnp.float32),
            pltpu.VMEM((q_rows, 128), jnp.float32),
            pltpu.SemaphoreType.DMA,
            pltpu.SemaphoreType.DMA,
        ],
    )
    def gather_kernel(table_hbm, idx_hbm, out_hbm, idx_v, rows_a, rows_b,
                      sem_a, sem_b):
        wid = lax.axis_index("s") * _NC + lax.axis_index("c")
        base = wid * b_per_w
        pltpu.sync_copy(idx_hbm.at[pl.ds(base, b_per_w)], idx_v)
        bufs = ((rows_a, sem_a), (rows_b, sem_b))

        def chunk_copies(qq):
            rows_v, sem = bufs[qq % 2]
            return [
                pltpu.make_async_copy(
                    table_hbm.at[idx_v.at[pl.ds(qq * q_rows + j * _CHUNK,
                                                _CHUNK)]],
                    rows_v.at[pl.ds(j * _CHUNK, _CHUNK)],
                    sem,
                )
                for j in range(n_chunks)
            ]

        for c in chunk_copies(0):
            c.start()
        for qq in range(1, n_q + 1):
            if qq <= n_q - 1:
                for c in chunk_copies(qq):
                    c.start()
            prev = qq - 1
            rows_v, _ = bufs[prev % 2]
            for c in chunk_copies(prev):
                c.wait()
            pltpu.sync_copy(rows_v,
                            out_hbm.at[pl.ds(base + prev * q_rows, q_rows)])

    return gather_kernel(table128, idx)


def _tc_transform(r2d, rows3, w_flat, rel_embed, batch, blk):
    n_blocks = batch // blk
    kdim = _NR * _D

    def body(r_ref, xh_ref, xp_ref, xn_ref, wf_ref, rel_ref,
             oh_ref, op_ref, on_ref, or_ref):
        rcol = r_ref[...]
        lane_rel = lax.broadcasted_iota(jnp.int32, (blk, kdim), 1) >> 6
        mask = lane_rel == rcol
        wf = wf_ref[...]

        def trans(x_ref, o_ref):
            x = x_ref[...][:, :_D]
            xt = jnp.concatenate([x] * _NR, axis=1)
            z = jnp.where(mask, xt, 0.0)
            o_ref[...] = jnp.dot(z, wf, preferred_element_type=jnp.float32)

        trans(xh_ref, oh_ref)
        trans(xp_ref, op_ref)
        trans(xn_ref, on_ref)

        onehot = (lax.broadcasted_iota(jnp.int32, (blk, _NR), 1)
                  == rcol).astype(jnp.float32)
        or_ref[...] = jnp.dot(onehot, rel_ref[...],
                              preferred_element_type=jnp.float32)

    out_block = pl.BlockSpec((blk, _D), lambda i: (i, 0))
    return pl.pallas_call(
        body,
        grid=(n_blocks,),
        in_specs=[
            pl.BlockSpec((blk, 1), lambda i: (i, 0)),
            pl.BlockSpec((blk, 128), lambda i: (i, 0)),
            pl.BlockSpec((blk, 128), lambda i: (i + n_blocks, 0)),
            pl.BlockSpec((blk, 128), lambda i: (i + 2 * n_blocks, 0)),
            pl.BlockSpec((kdim, _D), lambda i: (0, 0)),
            pl.BlockSpec((_NR, _D), lambda i: (0, 0)),
        ],
        out_specs=[out_block, out_block, out_block, out_block],
        out_shape=[jax.ShapeDtypeStruct((batch, _D), jnp.float32)] * 4,
    )(r2d, rows3, rows3, rows3, w_flat, rel_embed)


def kernel(h, r, pos_t, neg_t, entity_embed, relation_embed, W_r):
    batch = h.shape[0]
    table128 = jnp.pad(entity_embed, ((0, 0), (0, 128 - _D)))
    idx_all = jnp.concatenate([h, pos_t, neg_t]).astype(jnp.int32)
    rows3 = _sc_gather128(table128, idx_all)
    w_flat = W_r.reshape(_NR * _D, _D)
    r2d = r.astype(jnp.int32)[:, None]
    h_e, pos_t_e, neg_t_e, r_embed = _tc_transform(
        r2d, rows3, w_flat, relation_embed, batch, blk=1024)
    return (h_e, pos_t_e, neg_t_e, r_embed)
